# baseline (speedup 1.0000x reference)
import jax
import jax.numpy as jnp
from jax.experimental import pallas as pl
from jax.experimental.pallas import tpu as pltpu


def _dma_body(src_tab_ref, tgt_tab_ref, src_out_ref, tgt_out_ref, sem_ref):
    n = src_out_ref.shape[0]
    copies = []
    for b in range(n):
        copies.append(pltpu.make_async_copy(src_tab_ref, src_out_ref.at[b], sem_ref.at[b]))
        copies.append(pltpu.make_async_copy(tgt_tab_ref, tgt_out_ref.at[b], sem_ref.at[n + b]))
    for c in copies:
        c.start()
    for c in copies:
        c.wait()


def kernel(src, tgt, src_pos_table, tgt_pos_table):
    n = src.shape[0]
    src_len = src.shape[1]
    tgt_len = tgt.shape[1]
    embed = src_pos_table.shape[1]

    out = pl.pallas_call(
        _dma_body,
        in_specs=[
            pl.BlockSpec(memory_space=pl.ANY),
            pl.BlockSpec(memory_space=pl.ANY),
        ],
        out_specs=[
            pl.BlockSpec(memory_space=pl.ANY),
            pl.BlockSpec(memory_space=pl.ANY),
        ],
        out_shape=[
            jax.ShapeDtypeStruct((n, src_len, embed), src_pos_table.dtype),
            jax.ShapeDtypeStruct((n, tgt_len, embed), tgt_pos_table.dtype),
        ],
        scratch_shapes=[pltpu.SemaphoreType.DMA((2 * n,))],
    )(src_pos_table[:src_len], tgt_pos_table[:tgt_len])
    return (out[0], out[1])


# speedup vs baseline: 77.1989x; 77.1989x over previous
import jax
import jax.numpy as jnp
from jax.experimental import pallas as pl
from jax.experimental.pallas import tpu as pltpu

_ROWS = 1024


def _fanout_body(src_tab_ref, tgt_tab_ref, src_out_ref, tgt_out_ref, sem_ref):
    n = src_out_ref.shape[0]
    base = pl.program_id(0) * _ROWS
    copies = []
    for b in range(n):
        copies.append(
            pltpu.make_async_copy(
                src_tab_ref, src_out_ref.at[b, pl.ds(base, _ROWS)], sem_ref.at[b]
            )
        )
        copies.append(
            pltpu.make_async_copy(
                tgt_tab_ref, tgt_out_ref.at[b, pl.ds(base, _ROWS)], sem_ref.at[n + b]
            )
        )
    for c in copies:
        c.start()
    for c in copies:
        c.wait()


def kernel(src, tgt, src_pos_table, tgt_pos_table):
    n = src.shape[0]
    src_len = src.shape[1]
    tgt_len = tgt.shape[1]
    embed = src_pos_table.shape[1]

    out = pl.pallas_call(
        _fanout_body,
        grid=(src_len // _ROWS,),
        in_specs=[
            pl.BlockSpec((_ROWS, embed), lambda i: (i, 0)),
            pl.BlockSpec((_ROWS, embed), lambda i: (i, 0)),
        ],
        out_specs=[
            pl.BlockSpec(memory_space=pl.ANY),
            pl.BlockSpec(memory_space=pl.ANY),
        ],
        out_shape=[
            jax.ShapeDtypeStruct((n, src_len, embed), src_pos_table.dtype),
            jax.ShapeDtypeStruct((n, tgt_len, embed), tgt_pos_table.dtype),
        ],
        scratch_shapes=[pltpu.SemaphoreType.DMA((2 * n,))],
    )(src_pos_table[:src_len], tgt_pos_table[:tgt_len])
    return (out[0], out[1])
